# baseline (speedup 1.0000x reference)
import functools

import jax
import jax.numpy as jnp
import numpy as np
from jax.experimental import pallas as pl
from jax.experimental.pallas import tpu as pltpu

_BOUNDARY_MARGIN = 2.0
_BN = 1024


def _contract_last(a, b):
    return jax.lax.dot_general(a, b, (((1,), (1,)), ((), ())),
                               preferred_element_type=jnp.float32)


def _fused_body(x_ref, P3_ref, cents_ref, W1cT_ref, b1cT_ref, W2bdT_ref,
                RT_ref, S_ref, b2T_ref, outT_ref, *, n_exp):
    bn = x_ref.shape[0]
    x = x_ref[...]
    x3T = _contract_last(P3_ref[...], x)
    cd2 = jnp.zeros((n_exp, bn), dtype=jnp.float32)
    for i in range(3):
        d = x3T[i:i + 1, :] - cents_ref[:, i:i + 1]
        cd2 = cd2 + d * d
    cd = jnp.sqrt(cd2)
    inv = 1.0 / (cd + 1e-8)
    mind = jnp.min(cd, axis=0, keepdims=True)
    inv = jnp.where(cd > _BOUNDARY_MARGIN * mind, 0.0, inv)
    wT = inv / jnp.sum(inv, axis=0, keepdims=True)

    hT = _contract_last(W1cT_ref[...],
                        x.astype(jnp.bfloat16)) + b1cT_ref[...]
    hT = jnp.maximum(hT, 0.0).astype(jnp.bfloat16)
    M2T = jnp.dot(W2bdT_ref[...], hT,
                  preferred_element_type=jnp.float32)
    w_expT = jnp.dot(RT_ref[...], wT,
                     preferred_element_type=jnp.float32)
    outT = jnp.dot(S_ref[...], M2T * w_expT,
                   preferred_element_type=jnp.float32)
    outT = outT + jnp.dot(b2T_ref[...], wT,
                          preferred_element_type=jnp.float32)
    outT_ref[...] = outT


def kernel(x, centroids, W1, b1, W2, b2):
    N = x.shape[0]
    E, D_IN, H = W1.shape
    OUT = W2.shape[-1]
    EH, EO = E * H, E * OUT

    DX = x.shape[1]
    P3 = jnp.concatenate(
        [jnp.eye(3, dtype=jnp.float32),
         jnp.zeros((3, D_IN), dtype=jnp.float32)], axis=1)
    W1cT = jnp.transpose(W1, (1, 0, 2)).reshape(D_IN, EH).T
    W1cT = jnp.concatenate(
        [jnp.zeros((EH, 3), dtype=W1.dtype), W1cT],
        axis=1).astype(jnp.bfloat16)
    b1cT = b1.reshape(EH, 1)
    W2bdT = (jax.vmap(jnp.transpose)(W2)
             .reshape(EO, H))
    W2bdT = (W2bdT[:, None, :] *
             jnp.eye(E, dtype=W2.dtype).repeat(OUT, axis=0)[:, :, None]
             ).reshape(EO, EH).astype(jnp.bfloat16)
    RT = jnp.eye(E, dtype=jnp.float32).repeat(OUT, axis=0)
    S = jnp.tile(jnp.eye(OUT, dtype=jnp.float32), (1, E))
    b2T = b2.T

    grid = (N // _BN,)
    body = functools.partial(_fused_body, n_exp=E)
    outT = pl.pallas_call(
        body,
        grid=grid,
        in_specs=[
            pl.BlockSpec((_BN, DX), lambda i: (i, 0)),
            pl.BlockSpec((3, DX), lambda i: (0, 0)),
            pl.BlockSpec((E, 3), lambda i: (0, 0)),
            pl.BlockSpec((EH, DX), lambda i: (0, 0)),
            pl.BlockSpec((EH, 1), lambda i: (0, 0)),
            pl.BlockSpec((EO, EH), lambda i: (0, 0)),
            pl.BlockSpec((EO, E), lambda i: (0, 0)),
            pl.BlockSpec((OUT, EO), lambda i: (0, 0)),
            pl.BlockSpec((OUT, E), lambda i: (0, 0)),
        ],
        out_specs=pl.BlockSpec((OUT, _BN), lambda i: (0, i)),
        out_shape=jax.ShapeDtypeStruct((OUT, N), jnp.float32),
        compiler_params=pltpu.CompilerParams(
            dimension_semantics=("parallel",)),
    )(x, P3, centroids, W1cT, b1cT, W2bdT, RT, S, b2T)
    return outT.T


# speedup vs baseline: 1.1897x; 1.1897x over previous
import functools

import jax
import jax.numpy as jnp
import numpy as np
from jax.experimental import pallas as pl
from jax.experimental.pallas import tpu as pltpu

_BOUNDARY_MARGIN = 2.0
_BN = 2048


def _fused_body(x3T_ref, xfT_ref, cents_ref, W1cT_ref, b1cT_ref, W2bdT_ref,
                RT_ref, S_ref, b2T_ref, outT_ref, *, n_exp):
    bn = x3T_ref.shape[1]
    cd2 = jnp.zeros((n_exp, bn), dtype=jnp.float32)
    for i in range(3):
        d = x3T_ref[i:i + 1, :] - cents_ref[:, i:i + 1]
        cd2 = cd2 + d * d
    cd = jnp.sqrt(cd2)
    inv = 1.0 / (cd + 1e-8)
    mind = jnp.min(cd, axis=0, keepdims=True)
    inv = jnp.where(cd > _BOUNDARY_MARGIN * mind, 0.0, inv)
    wT = inv / jnp.sum(inv, axis=0, keepdims=True)

    hT = jnp.dot(W1cT_ref[...], xfT_ref[...],
                 preferred_element_type=jnp.float32) + b1cT_ref[...]
    hT = jnp.maximum(hT, 0.0).astype(jnp.bfloat16)
    M2T = jnp.dot(W2bdT_ref[...], hT,
                  preferred_element_type=jnp.float32)
    w_expT = jnp.dot(RT_ref[...], wT,
                     preferred_element_type=jnp.float32)
    outT = jnp.dot(S_ref[...], M2T * w_expT,
                   preferred_element_type=jnp.float32)
    outT = outT + jnp.dot(b2T_ref[...], wT,
                          preferred_element_type=jnp.float32)
    outT_ref[...] = outT


def kernel(x, centroids, W1, b1, W2, b2):
    N = x.shape[0]
    E, D_IN, H = W1.shape
    OUT = W2.shape[-1]
    EH, EO = E * H, E * OUT

    x3T = x[:, :3].T
    xfT = x[:, 3:].astype(jnp.bfloat16).T
    W1cT = (jnp.transpose(W1, (1, 0, 2)).reshape(D_IN, EH)
            .T.astype(jnp.bfloat16))
    b1cT = b1.reshape(EH, 1)
    W2bdT = (jax.vmap(jnp.transpose)(W2)
             .reshape(EO, H))
    W2bdT = (W2bdT[:, None, :] *
             jnp.eye(E, dtype=W2.dtype).repeat(OUT, axis=0)[:, :, None]
             ).reshape(EO, EH).astype(jnp.bfloat16)
    RT = jnp.eye(E, dtype=jnp.float32).repeat(OUT, axis=0)
    S = jnp.tile(jnp.eye(OUT, dtype=jnp.float32), (1, E))
    b2T = b2.T

    grid = (N // _BN,)
    body = functools.partial(_fused_body, n_exp=E)
    outT = pl.pallas_call(
        body,
        grid=grid,
        in_specs=[
            pl.BlockSpec((3, _BN), lambda i: (0, i)),
            pl.BlockSpec((D_IN, _BN), lambda i: (0, i)),
            pl.BlockSpec((E, 3), lambda i: (0, 0)),
            pl.BlockSpec((EH, D_IN), lambda i: (0, 0)),
            pl.BlockSpec((EH, 1), lambda i: (0, 0)),
            pl.BlockSpec((EO, EH), lambda i: (0, 0)),
            pl.BlockSpec((EO, E), lambda i: (0, 0)),
            pl.BlockSpec((OUT, EO), lambda i: (0, 0)),
            pl.BlockSpec((OUT, E), lambda i: (0, 0)),
        ],
        out_specs=pl.BlockSpec((OUT, _BN), lambda i: (0, i)),
        out_shape=jax.ShapeDtypeStruct((OUT, N), jnp.float32),
        compiler_params=pltpu.CompilerParams(
            dimension_semantics=("parallel",)),
    )(x3T, xfT, centroids, W1cT, b1cT, W2bdT, RT, S, b2T)
    return outT.T


# speedup vs baseline: 1.2630x; 1.0616x over previous
import functools

import jax
import jax.numpy as jnp
from jax.experimental import pallas as pl
from jax.experimental.pallas import tpu as pltpu

_BOUNDARY_MARGIN = 2.0
_BN = 2048


def _fused_body(x3T_ref, xfT_ref, cents_ref, W1cT_ref, W2cT_ref, b2T_ref,
                outT_ref, *, n_exp, hid, n_out):
    bn = x3T_ref.shape[1]
    cd2 = jnp.zeros((n_exp, bn), dtype=jnp.float32)
    for i in range(3):
        d = x3T_ref[i:i + 1, :] - cents_ref[:, i:i + 1]
        cd2 = cd2 + d * d
    cd = jnp.sqrt(cd2)
    inv = 1.0 / (cd + 1e-8)
    mind = jnp.min(cd, axis=0, keepdims=True)
    inv = jnp.where(cd > _BOUNDARY_MARGIN * mind, 0.0, inv)
    wT = inv / jnp.sum(inv, axis=0, keepdims=True)

    xfT = xfT_ref[...]
    outT = jnp.dot(b2T_ref[...], wT,
                   preferred_element_type=jnp.float32)
    for e in range(n_exp):
        hT = jnp.dot(W1cT_ref[e * hid:(e + 1) * hid, :], xfT,
                     preferred_element_type=jnp.float32).astype(jnp.bfloat16)
        hT = jnp.maximum(hT, jnp.bfloat16(0.0))
        m2 = jnp.dot(W2cT_ref[e * n_out:(e + 1) * n_out, :], hT,
                     preferred_element_type=jnp.float32)
        outT = outT + m2 * wT[e:e + 1, :]
    outT_ref[...] = outT


def kernel(x, centroids, W1, b1, W2, b2):
    N = x.shape[0]
    E, D_IN, H = W1.shape
    OUT = W2.shape[-1]
    EH, EO = E * H, E * OUT

    x3T = x[:, :3].T
    xfT = jnp.concatenate(
        [x[:, 3:].astype(jnp.bfloat16).T,
         jnp.ones((1, N), dtype=jnp.bfloat16)], axis=0)
    W1cT = jnp.concatenate(
        [jnp.transpose(W1, (0, 2, 1)).reshape(EH, D_IN),
         b1.reshape(EH, 1)], axis=1).astype(jnp.bfloat16)
    W2cT = jnp.transpose(W2, (0, 2, 1)).reshape(EO, H).astype(jnp.bfloat16)
    b2T = b2.T

    grid = (N // _BN,)
    body = functools.partial(_fused_body, n_exp=E, hid=H, n_out=OUT)
    outT = pl.pallas_call(
        body,
        grid=grid,
        in_specs=[
            pl.BlockSpec((3, _BN), lambda i: (0, i)),
            pl.BlockSpec((D_IN + 1, _BN), lambda i: (0, i)),
            pl.BlockSpec((E, 3), lambda i: (0, 0)),
            pl.BlockSpec((EH, D_IN + 1), lambda i: (0, 0)),
            pl.BlockSpec((EO, H), lambda i: (0, 0)),
            pl.BlockSpec((OUT, E), lambda i: (0, 0)),
        ],
        out_specs=pl.BlockSpec((OUT, _BN), lambda i: (0, i)),
        out_shape=jax.ShapeDtypeStruct((OUT, N), jnp.float32),
        compiler_params=pltpu.CompilerParams(
            dimension_semantics=("parallel",)),
    )(x3T, xfT, centroids, W1cT, W2cT, b2T)
    return outT.T
